# Initial kernel scaffold; baseline (speedup 1.0000x reference)
import jax
import jax.numpy as jnp
from jax.experimental import pallas as pl


def kernel(x, word_pos, table):
    raise NotImplementedError("write your pallas kernel here")


# speedup vs baseline: 16.1822x; 16.1822x over previous
import functools

import jax
import jax.numpy as jnp
from jax import lax
from jax.experimental import pallas as pl
from jax.experimental.pallas import tpu as pltpu
from jax.experimental.pallas import tpu_sc as plsc

VOC = 1_000_000
D = 32
B = 16384
L = 200

_info = plsc.get_sparse_core_info()
NC = _info.num_cores
NS = _info.num_subcores
NW = NC * NS

RW = B // NW
SCH_ROWS = 8
NSUP = RW // SCH_ROWS
HALF = L // 2
GPS = SCH_ROWS * 2
CHUNK = SCH_ROWS * L

_mesh = plsc.VectorSubcoreMesh(core_axis_name="c", subcore_axis_name="s")


@functools.partial(
    pl.kernel,
    mesh=_mesh,
    compiler_params=pltpu.CompilerParams(use_tc_tiling_on_sc=False),
    out_type=jax.ShapeDtypeStruct((B, D), jnp.float32),
    scratch_types=[
        pltpu.VMEM((2, GPS, HALF), jnp.int32),
        pltpu.VMEM((2, CHUNK, D), jnp.float32),
        pltpu.VMEM((RW, D), jnp.float32),
        pltpu.SemaphoreType.DMA,
        pltpu.SemaphoreType.DMA,
    ],
)
def _emb(x2_hbm, table_hbm, out_hbm, idx_v, rows_v, out_v, sem0, sem1):
    wid = lax.axis_index("s") * NC + lax.axis_index("c")
    xbase = wid * (RW * 2)
    obase = wid * RW
    sems = (sem0, sem1)

    def load_and_fire(s, b):
        pltpu.sync_copy(
            x2_hbm.at[pl.ds(xbase + s * (SCH_ROWS * 2), SCH_ROWS * 2)],
            idx_v.at[b],
        )
        for g in range(GPS):
            pltpu.make_async_copy(
                table_hbm.at[idx_v.at[b, g]],
                rows_v.at[b, pl.ds(g * HALF, HALF)],
                sems[b],
            ).start()

    def drain(b):
        pltpu.make_async_copy(
            table_hbm.at[pl.ds(0, CHUNK)],
            rows_v.at[b],
            sems[b],
        ).wait()

    load_and_fire(0, 0)
    load_and_fire(1, 1)

    inv_l = jnp.float32(1.0 / L)

    def outer(i, carry):
        s0 = i * 2
        for b in range(2):
            s = s0 + b
            drain(b)
            for o in range(SCH_ROWS):
                def step(j, acc, _o=o, _b=b):
                    a0, a1 = acc
                    r = _o * L + j
                    a0 = a0 + rows_v[_b, r, pl.ds(0, 16)]
                    a1 = a1 + rows_v[_b, r, pl.ds(16, 16)]
                    return a0, a1

                z = jnp.zeros((16,), jnp.float32)
                a0, a1 = lax.fori_loop(0, L, step, (z, z), unroll=10)
                orow = s * SCH_ROWS + o
                out_v[orow, pl.ds(0, 16)] = a0 * inv_l
                out_v[orow, pl.ds(16, 16)] = a1 * inv_l

            @pl.when(s + 2 < NSUP)
            def _(s=s, b=b):
                load_and_fire(s + 2, b)
        return carry

    lax.fori_loop(0, NSUP // 2, outer, 0)
    pltpu.sync_copy(out_v, out_hbm.at[pl.ds(obase, RW)])


def kernel(x, word_pos, table):
    del word_pos
    x2 = x.reshape(2 * B, HALF).astype(jnp.int32)
    return _emb(x2, table)
